# Initial kernel scaffold; baseline (speedup 1.0000x reference)
import jax
import jax.numpy as jnp
from jax.experimental import pallas as pl


def kernel(x, edge_index, W_self1, W_neigh1, b1, W_self2, W_neigh2, b2):
    raise NotImplementedError("write your pallas kernel here")


# speedup vs baseline: 10.9289x; 10.9289x over previous
import functools
import jax
import jax.numpy as jnp
from jax import lax
from jax.experimental import pallas as pl
from jax.experimental.pallas import tpu as pltpu
from jax.experimental.pallas import tpu_sc as plsc

N = 10000
E = 320000
D_IN = 128
D_H = 128
D_OUT = 40

NC = 2
NS = 16
NW = NC * NS
EPW = 10240
E_PAD = NW * EPW
N_PAD = 10112
DSH = 128


def _sc_seg_sum(d_feat, with_deg, ch, g_sz, ngrp):
  mesh = plsc.VectorSubcoreMesh(core_axis_name="c", subcore_axis_name="s",
                                num_cores=NC, num_subcores=NS)
  out_type = [jax.ShapeDtypeStruct((NC * N, d_feat), jnp.float32)]
  if with_deg:
    out_type.append(jax.ShapeDtypeStruct((NW, N_PAD // DSH, DSH), jnp.float32))
  scratch = [
      pltpu.VMEM((g_sz, ch), jnp.int32),
      pltpu.VMEM((g_sz, ch), jnp.int32),
      pltpu.VMEM((ch, d_feat), jnp.float32),
      pltpu.VMEM((ch, d_feat), jnp.float32),
      pltpu.VMEM((N_PAD // DSH, DSH), jnp.float32),
      pltpu.VMEM_SHARED((N_PAD, d_feat), jnp.float32),
      pltpu.SemaphoreType.DMA,
      pltpu.SemaphoreType.DMA,
      pltpu.SemaphoreType.DMA,
      pltpu.SemaphoreType.DMA,
  ]

  def body(src_hbm, dst_hbm, table_hbm, zrows_hbm, zdeg_hbm,
           out_hbm, *rest):
    if with_deg:
      deg_hbm = rest[0]
      rest = rest[1:]
    sidx, didx, rows0, rows1, degv, acc, g0, g1, s0, s1 = rest
    rows = (rows0, rows1)
    gsem = (g0, g1)
    ssem = (s0, s1)
    c = lax.axis_index("c")
    s = lax.axis_index("s")
    wid = c * NS + s

    zslab = N_PAD // NS
    pltpu.sync_copy(zrows_hbm.at[pl.ds(s * zslab, zslab)],
                    acc.at[pl.ds(s * zslab, zslab)])
    if with_deg:
      pltpu.sync_copy(zdeg_hbm, degv)

    plsc.subcore_barrier()
    ones16 = jnp.full((16,), 1.0, jnp.float32)

    def deg_update(t):
      for k in range(ch // 16):
        idx16 = didx[t, pl.ds(k * 16, 16)]
        hi = lax.shift_right_logical(idx16, 7)
        lo = lax.bitwise_and(idx16, DSH - 1)
        plsc.addupdate_scatter(degv, [hi, lo], ones16)

    def group(g, _):
      pltpu.sync_copy(src_hbm.at[wid, pl.ds(g * g_sz, g_sz)], sidx)
      pltpu.sync_copy(dst_hbm.at[wid, pl.ds(g * g_sz, g_sz)], didx)
      gd = [None, None]
      sd = [None, None]
      gd[0] = pltpu.async_copy(table_hbm.at[sidx.at[0]], rows[0], gsem[0])
      for t in range(1, g_sz):
        b, pb = t % 2, (t - 1) % 2
        if t >= 2:
          sd[b].wait()
        gd[b] = pltpu.async_copy(table_hbm.at[sidx.at[t]], rows[b], gsem[b])
        gd[pb].wait()
        sd[pb] = pltpu.async_copy(rows[pb], acc.at[didx.at[t - 1]], ssem[pb],
                                  add=True)
        if with_deg:
          deg_update(t - 1)
      lb, plb = (g_sz - 1) % 2, (g_sz - 2) % 2
      gd[lb].wait()
      sd[lb] = pltpu.async_copy(rows[lb], acc.at[didx.at[g_sz - 1]], ssem[lb],
                                add=True)
      if with_deg:
        deg_update(g_sz - 1)
      sd[plb].wait()
      sd[lb].wait()
      return _

    lax.fori_loop(0, ngrp, group, None)
    plsc.subcore_barrier()

    wslab = 624
    pltpu.sync_copy(acc.at[pl.ds(s * wslab, wslab)],
                    out_hbm.at[pl.ds(c * N + s * wslab, wslab)])
    @pl.when(s == NS - 1)
    def _():
      rem = N - NS * wslab
      pltpu.sync_copy(acc.at[pl.ds(NS * wslab, rem)],
                      out_hbm.at[pl.ds(c * N + NS * wslab, rem)])
    if with_deg:
      pltpu.sync_copy(degv, deg_hbm.at[wid])

  return pl.kernel(body, out_type=out_type, mesh=mesh, scratch_types=scratch,
                   compiler_params=pltpu.CompilerParams(
                       needs_layout_passes=False),
                   name=f"sc_seg_sum_{d_feat}")


CH1, G1, NGRP1 = 80, 16, 8
_seg_sum_deg = _sc_seg_sum(D_IN, True, CH1, G1, NGRP1)
_seg_sum_h = _sc_seg_sum(D_H, False, CH1, G1, NGRP1)

_R = 1000


def _tc_rcp_body(dg_ref, rcp_ref):
  deg = jnp.sum(dg_ref[...], axis=0)
  rcp_ref[...] = 1.0 / jnp.maximum(deg, 1.0)


def _tc_rcp(dg):
  nb = N_PAD // DSH
  return pl.pallas_call(
      _tc_rcp_body,
      in_specs=[pl.BlockSpec((NW, nb, DSH), lambda: (0, 0, 0))],
      out_specs=pl.BlockSpec((nb, DSH), lambda: (0, 0)),
      out_shape=jax.ShapeDtypeStruct((nb, DSH), jnp.float32),
      name="tc_rcp",
  )(dg)


def _tc_mid_body(x_ref, p1_ref, rcp_ref, ws1_ref, wn1_ref, b1_ref, h_ref):
  agg = p1_ref[0] + p1_ref[1]
  hn = agg * rcp_ref[...]
  x = x_ref[...]
  h = jnp.dot(x, ws1_ref[...], preferred_element_type=jnp.float32)
  h += jnp.dot(hn, wn1_ref[...], preferred_element_type=jnp.float32)
  h_ref[...] = jnp.maximum(h + b1_ref[...], 0.0)


def _tc_mid(x, p1, rcp, ws1, wn1, b1):
  grid = (N // _R,)
  return pl.pallas_call(
      _tc_mid_body,
      grid=grid,
      in_specs=[
          pl.BlockSpec((_R, D_IN), lambda i: (i, 0)),
          pl.BlockSpec((NC, _R, D_H), lambda i: (0, i, 0)),
          pl.BlockSpec((_R, 1), lambda i: (i, 0)),
          pl.BlockSpec((D_IN, D_H), lambda i: (0, 0)),
          pl.BlockSpec((D_IN, D_H), lambda i: (0, 0)),
          pl.BlockSpec((1, D_H), lambda i: (0, 0)),
      ],
      out_specs=pl.BlockSpec((_R, D_H), lambda i: (i, 0)),
      out_shape=jax.ShapeDtypeStruct((N, D_H), jnp.float32),
      name="tc_mid",
  )(x, p1, rcp, ws1, wn1, b1)


def _tc_final_body(h_ref, p2_ref, rcp_ref, ws2_ref, wn2_ref, b2_ref, out_ref):
  agg = p2_ref[0] + p2_ref[1]
  hn = agg * rcp_ref[...]
  out = jnp.dot(h_ref[...], ws2_ref[...], preferred_element_type=jnp.float32)
  out += jnp.dot(hn, wn2_ref[...], preferred_element_type=jnp.float32)
  out_ref[...] = out + b2_ref[...]


def _tc_final(h, p2, rcp, ws2, wn2, b2):
  grid = (N // _R,)
  return pl.pallas_call(
      _tc_final_body,
      grid=grid,
      in_specs=[
          pl.BlockSpec((_R, D_H), lambda i: (i, 0)),
          pl.BlockSpec((NC, _R, D_H), lambda i: (0, i, 0)),
          pl.BlockSpec((_R, 1), lambda i: (i, 0)),
          pl.BlockSpec((D_H, D_OUT), lambda i: (0, 0)),
          pl.BlockSpec((D_H, D_OUT), lambda i: (0, 0)),
          pl.BlockSpec((1, D_OUT), lambda i: (0, 0)),
      ],
      out_specs=pl.BlockSpec((_R, D_OUT), lambda i: (i, 0)),
      out_shape=jax.ShapeDtypeStruct((N, D_OUT), jnp.float32),
      name="tc_final",
  )(h, p2, rcp, ws2, wn2, b2)


@jax.jit
def kernel(x, edge_index, W_self1, W_neigh1, b1, W_self2, W_neigh2, b2):
  src = edge_index[0]
  dst = edge_index[1]
  pad = E_PAD - E
  pad_i = jnp.arange(pad, dtype=jnp.int32)
  src_flat = jnp.concatenate([src, pad_i % N])
  dst_flat = jnp.concatenate([dst, N + pad_i % (N_PAD - N)])
  src_l1 = src_flat.reshape(NW, EPW // CH1, CH1)
  dst_l1 = dst_flat.reshape(NW, EPW // CH1, CH1)
  zrows = jnp.zeros((N_PAD, D_IN), jnp.float32)
  zdeg = jnp.zeros((N_PAD // DSH, DSH), jnp.float32)

  p1, dg = _seg_sum_deg(src_l1, dst_l1, x, zrows, zdeg)
  p1 = p1.reshape(NC, N, D_H)

  rcp = _tc_rcp(dg).reshape(N_PAD, 1)

  b1r = b1.reshape(1, D_H)
  b2r = b2.reshape(1, D_OUT)
  h = _tc_mid(x, p1, rcp, W_self1, W_neigh1, b1r)

  p2 = _seg_sum_h(src_l1, dst_l1, h, zrows, zdeg)[0]
  p2 = p2.reshape(NC, N, D_H)

  return _tc_final(h, p2, rcp, W_self2, W_neigh2, b2r)
